# Initial kernel scaffold; baseline (speedup 1.0000x reference)
import jax
import jax.numpy as jnp
from jax.experimental import pallas as pl


def kernel(z_concat, G1, G2, W1, b1, W2, b2, W3, b3):
    raise NotImplementedError("write your pallas kernel here")


# speedup vs baseline: 1.3605x; 1.3605x over previous
import jax
import jax.numpy as jnp
from jax.experimental import pallas as pl
from jax.experimental.pallas import tpu as pltpu

_N = 4096
_D = 4096
_H = 1024


def _gate_body(z_ref, w1_ref, b1_ref, w2_ref, b2_ref, w3_ref, b3_ref, g0_ref):
    z = z_ref[...]
    mu = jnp.mean(z, axis=1, keepdims=True)
    zc = z - mu
    var = jnp.mean(zc * zc, axis=1, keepdims=True)
    zn = zc * jax.lax.rsqrt(var + 1e-5)
    h1 = jnp.dot(zn, w1_ref[...], preferred_element_type=jnp.float32) + b1_ref[...]
    h1 = jnp.maximum(h1, 0.0)
    h2 = jnp.dot(h1, w2_ref[...], preferred_element_type=jnp.float32) + b2_ref[...]
    h2 = jnp.where(h2 > 0, h2, 0.01 * h2)
    x = jnp.dot(h2, w3_ref[...], preferred_element_type=jnp.float32) + b3_ref[...]
    m = jnp.max(x, axis=1, keepdims=True)
    e = jnp.exp(x - m)
    g0_ref[...] = e / jnp.sum(e, axis=1, keepdims=True)


def _fuse_body(g1_ref, g2_ref, g0all_ref, g0row_ref, gf_ref, gw_ref):
    g1 = g1_ref[...]
    s = jnp.dot(g1, g0all_ref[...], preferred_element_type=jnp.float32)
    gw = 0.7 * g0row_ref[...] + 0.3 * s
    gw_ref[...] = gw
    gf_ref[...] = g1 * gw[:, 0:1] + g2_ref[...] * gw[:, 1:2]


def kernel(z_concat, G1, G2, W1, b1, W2, b2, W3, b3):
    w3s = W3 * 8.0
    b3s = b3 * 8.0 + jnp.array([5.0, 0.0], dtype=jnp.float32)
    b1r = b1.reshape(1, _H)
    b2r = b2.reshape(1, 64)
    b3r = b3s.reshape(1, 2)

    br_a = 512
    g0 = pl.pallas_call(
        _gate_body,
        grid=(_N // br_a,),
        in_specs=[
            pl.BlockSpec((br_a, _D), lambda i: (i, 0)),
            pl.BlockSpec((_D, _H), lambda i: (0, 0)),
            pl.BlockSpec((1, _H), lambda i: (0, 0)),
            pl.BlockSpec((_H, 64), lambda i: (0, 0)),
            pl.BlockSpec((1, 64), lambda i: (0, 0)),
            pl.BlockSpec((64, 2), lambda i: (0, 0)),
            pl.BlockSpec((1, 2), lambda i: (0, 0)),
        ],
        out_specs=pl.BlockSpec((br_a, 2), lambda i: (i, 0)),
        out_shape=jax.ShapeDtypeStruct((_N, 2), jnp.float32),
        compiler_params=pltpu.CompilerParams(
            dimension_semantics=("arbitrary",),
        ),
    )(z_concat, W1, b1r, W2, b2r, w3s, b3r)

    br_b = 256
    gf, gw = pl.pallas_call(
        _fuse_body,
        grid=(_N // br_b,),
        in_specs=[
            pl.BlockSpec((br_b, _N), lambda i: (i, 0)),
            pl.BlockSpec((br_b, _N), lambda i: (i, 0)),
            pl.BlockSpec((_N, 2), lambda i: (0, 0)),
            pl.BlockSpec((br_b, 2), lambda i: (i, 0)),
        ],
        out_specs=[
            pl.BlockSpec((br_b, _N), lambda i: (i, 0)),
            pl.BlockSpec((br_b, 2), lambda i: (i, 0)),
        ],
        out_shape=[
            jax.ShapeDtypeStruct((_N, _N), jnp.float32),
            jax.ShapeDtypeStruct((_N, 2), jnp.float32),
        ],
        compiler_params=pltpu.CompilerParams(
            dimension_semantics=("arbitrary",),
        ),
    )(G1, G2, g0, g0)

    return (gf, gw)
